# Initial kernel scaffold; baseline (speedup 1.0000x reference)
import jax
import jax.numpy as jnp
from jax.experimental import pallas as pl


def kernel(x, W1, b1, W2, b2, Wu, bu):
    raise NotImplementedError("write your pallas kernel here")


# speedup vs baseline: 14.4007x; 14.4007x over previous
import functools

import jax
import jax.numpy as jnp
from jax import lax
from jax.experimental import pallas as pl
from jax.experimental.pallas import tpu as pltpu
from jax.experimental.pallas import tpu_sc as plsc

_B, _C, _H, _W = 4, 32, 64, 64
_K = 8
_VS = 8
_P = _H * _W
_COUT = 64
_HID = 4 * _C
_ADIN = 2 + _C
_NIDX = _B * _P * _K * _VS
_NW = 32
_CH = 128
_PER_W = _NIDX // _NW
_MIN_SIGMA = 0.05
_SIGMA_SCALE = 0.05
_SIGMA_BOOST = 2.0
_REGION = 12


def _stage_a_body(xin_ref, w1_ref, b1_ref, w2_ref, b2_ref, mids_ref,
                  giy0_ref, gix0_ref, giy1_ref, gix1_ref,
                  roy0_ref, rox0_ref, roy1_ref, rox1_ref,
                  ids_ref, wts_ref):
    b = pl.program_id(0)
    xin = xin_ref[0]
    hid = jnp.maximum(
        jnp.dot(xin, w1_ref[...], preferred_element_type=jnp.float32)
        + b1_ref[...], 0.0)
    par = (jnp.dot(hid, w2_ref[...], preferred_element_type=jnp.float32)
           + b2_ref[...])
    hwf = jnp.float32(_H)
    my = jnp.mod(mids_ref[:, 0:1] + par[:, 0:_K], hwf)
    mx = jnp.mod(mids_ref[:, 1:2] + par[:, _K:2 * _K], hwf)
    sig = ((jax.nn.softplus(par[:, 2 * _K:3 * _K] + _SIGMA_BOOST)
            + _MIN_SIGMA) * (hwf * _SIGMA_SCALE))
    fyi = jnp.floor(my).astype(jnp.int32)
    fxi = jnp.floor(mx).astype(jnp.int32)

    iys = []
    ixs = []
    for dy, dx in ((0, 0), (0, 1), (1, 0), (1, 1)):
        iys.append(jnp.mod(fyi + dy, _H))
        ixs.append(jnp.mod(fxi + dx, _W))
    iys.append(giy0_ref[0])
    ixs.append(gix0_ref[0])
    iys.append(giy1_ref[0])
    ixs.append(gix1_ref[0])
    iys.append(jnp.mod(fyi + roy0_ref[0], _H))
    ixs.append(jnp.mod(fxi + rox0_ref[0], _W))
    iys.append(jnp.mod(fyi + roy1_ref[0], _H))
    ixs.append(jnp.mod(fxi + rox1_ref[0], _W))

    ids = [iy * _W + ix for iy, ix in zip(iys, ixs)]
    props = []
    for v in range(_VS):
        dyv = (iys[v].astype(jnp.float32) - my) / sig
        dxv = (ixs[v].astype(jnp.float32) - mx) / sig
        pr = jnp.exp(-0.5 * (dyv * dyv + dxv * dxv))
        if v > 0:
            dup = ids[0] == ids[v]
            for u in range(1, v):
                dup = jnp.logical_or(dup, ids[u] == ids[v])
            pr = jnp.where(dup, 0.0, pr)
        props.append(pr)
    tot = props[0]
    for v in range(1, _VS):
        tot = tot + props[v]
    base = b * _P
    ids_ref[0] = jnp.concatenate([i + base for i in ids], axis=1)
    wts_ref[0] = jnp.concatenate([p / tot for p in props], axis=1)


def _stage_c_body(g_ref, w_ref, wu_ref, bu_ref, out_ref):
    g = g_ref[...]
    wv = w_ref[...]
    feats = []
    for k in range(_K):
        acc = None
        for v in range(_VS):
            j = v * _K + k
            term = g[:, j * _C:(j + 1) * _C] * wv[:, j:j + 1]
            acc = term if acc is None else acc + term
        feats.append(acc)
    f = jnp.concatenate(feats, axis=1)
    out_ref[...] = (jnp.dot(f, wu_ref[...], preferred_element_type=jnp.float32)
                    + bu_ref[...])


def _sc_gather_body(table_hbm, idx_hbm, out_hbm, idx_v, rows_v, sem):
    wid = lax.axis_index("s") * 2 + lax.axis_index("c")
    wbase = wid * _PER_W

    def step(i, carry):
        base = wbase + i * _CH
        pltpu.sync_copy(idx_hbm.at[pl.ds(base, _CH)], idx_v)
        pltpu.async_copy(table_hbm.at[idx_v], rows_v, sem).wait()
        pltpu.sync_copy(rows_v, out_hbm.at[pl.ds(base, _CH)])
        return carry

    lax.fori_loop(0, _PER_W // _CH, step, 0)


@jax.jit
def kernel(x, W1, b1, W2, b2, Wu, bu):
    b, c, h, w = x.shape
    hwf = jnp.float32(h)

    xt = jnp.transpose(x, (0, 2, 3, 1))
    table = xt.reshape(b * _P, c)
    ys = jnp.linspace(0.0, 1.0, h)
    xs = jnp.linspace(0.0, 1.0, w)
    gy, gx = jnp.meshgrid(ys, xs, indexing="ij")
    coords = jnp.stack([gy, gx], axis=-1).reshape(_P, 2)
    mids = coords * (hwf - 1.0)
    xin = jnp.concatenate(
        [xt.reshape(b, _P, c),
         jnp.broadcast_to(coords[None], (b, _P, 2))], axis=-1)

    perm = jnp.array(
        tuple(range(0, 2 * _K, 2)) + tuple(range(1, 2 * _K, 2))
        + tuple(range(2 * _K, 3 * _K)), dtype=jnp.int32)
    W2p = W2[:, perm]
    b2p = b2[perm].reshape(1, 3 * _K)
    b1r = b1.reshape(1, _HID)
    bur = bu.reshape(1, _COUT)

    kg, kr = jax.random.split(jax.random.key(42))
    gidx = jax.random.randint(kg, (b, h, w, _K, 2, 2), 0, jnp.array([h, w]))
    roff = jax.random.randint(
        kr, (b, h, w, _K, 2, 2), 0,
        jnp.array([_REGION, _REGION])) - jnp.array([_REGION, _REGION]) // 2
    g4 = gidx.reshape(b, _P, _K, 2, 2).astype(jnp.int32)
    r4 = roff.reshape(b, _P, _K, 2, 2).astype(jnp.int32)
    giy0, gix0 = g4[..., 0, 0], g4[..., 0, 1]
    giy1, gix1 = g4[..., 1, 0], g4[..., 1, 1]
    roy0, rox0 = r4[..., 0, 0], r4[..., 0, 1]
    roy1, rox1 = r4[..., 1, 0], r4[..., 1, 1]

    pb = 512
    bspec = pl.BlockSpec((1, pb, _K), lambda i, j: (i, j, 0))
    ids, wts = pl.pallas_call(
        _stage_a_body,
        grid=(b, _P // pb),
        in_specs=[
            pl.BlockSpec((1, pb, _ADIN), lambda i, j: (i, j, 0)),
            pl.BlockSpec((_ADIN, _HID), lambda i, j: (0, 0)),
            pl.BlockSpec((1, _HID), lambda i, j: (0, 0)),
            pl.BlockSpec((_HID, 3 * _K), lambda i, j: (0, 0)),
            pl.BlockSpec((1, 3 * _K), lambda i, j: (0, 0)),
            pl.BlockSpec((pb, 2), lambda i, j: (j, 0)),
            bspec, bspec, bspec, bspec, bspec, bspec, bspec, bspec,
        ],
        out_specs=[
            pl.BlockSpec((1, pb, _VS * _K), lambda i, j: (i, j, 0)),
            pl.BlockSpec((1, pb, _VS * _K), lambda i, j: (i, j, 0)),
        ],
        out_shape=[
            jax.ShapeDtypeStruct((b, _P, _VS * _K), jnp.int32),
            jax.ShapeDtypeStruct((b, _P, _VS * _K), jnp.float32),
        ],
    )(xin, W1, b1r, W2p, b2p, mids,
      giy0, gix0, giy1, gix1, roy0, rox0, roy1, rox1)

    idx_flat = ids.reshape(_NIDX)
    mesh = plsc.VectorSubcoreMesh(core_axis_name="c", subcore_axis_name="s")
    gather = functools.partial(
        pl.kernel, _sc_gather_body, mesh=mesh,
        compiler_params=pltpu.CompilerParams(use_tc_tiling_on_sc=False),
        out_type=jax.ShapeDtypeStruct((_NIDX, _C), jnp.float32),
        scratch_types=[
            pltpu.VMEM((_CH,), jnp.int32),
            pltpu.VMEM((_CH, _C), jnp.float32),
            pltpu.SemaphoreType.DMA,
        ],
    )()
    gathered = gather(table, idx_flat)

    pc = 256
    g2 = gathered.reshape(b * _P, _VS * _K * _C)
    w2d = wts.reshape(b * _P, _VS * _K)
    out = pl.pallas_call(
        _stage_c_body,
        grid=(b * _P // pc,),
        in_specs=[
            pl.BlockSpec((pc, _VS * _K * _C), lambda i: (i, 0)),
            pl.BlockSpec((pc, _VS * _K), lambda i: (i, 0)),
            pl.BlockSpec((_K * _C, _COUT), lambda i: (0, 0)),
            pl.BlockSpec((1, _COUT), lambda i: (0, 0)),
        ],
        out_specs=pl.BlockSpec((pc, _COUT), lambda i: (i, 0)),
        out_shape=jax.ShapeDtypeStruct((b * _P, _COUT), jnp.float32),
    )(g2, w2d, Wu, bur)

    return jnp.transpose(out.reshape(b, h, w, _COUT), (0, 3, 1, 2))


# speedup vs baseline: 17.4035x; 1.2085x over previous
import functools

import jax
import jax.numpy as jnp
from jax import lax
from jax.experimental import pallas as pl
from jax.experimental.pallas import tpu as pltpu
from jax.experimental.pallas import tpu_sc as plsc

_B, _C, _H, _W = 4, 32, 64, 64
_K = 8
_VS = 8
_P = _H * _W
_COUT = 64
_HID = 4 * _C
_ADIN = 2 + _C
_NIDX = _B * _P * _K * _VS
_NW = 32
_CH = 512
_PER_W = _NIDX // _NW
_MIN_SIGMA = 0.05
_SIGMA_SCALE = 0.05
_SIGMA_BOOST = 2.0
_REGION = 12


def _stage_a_body(xin_ref, w1_ref, b1_ref, w2_ref, b2_ref, mids_ref,
                  giy0_ref, gix0_ref, giy1_ref, gix1_ref,
                  roy0_ref, rox0_ref, roy1_ref, rox1_ref,
                  ids_ref, wts_ref):
    b = pl.program_id(0)
    xin = xin_ref[0]
    hid = jnp.maximum(
        jnp.dot(xin, w1_ref[...], preferred_element_type=jnp.float32)
        + b1_ref[...], 0.0)
    par = (jnp.dot(hid, w2_ref[...], preferred_element_type=jnp.float32)
           + b2_ref[...])
    hwf = jnp.float32(_H)
    my = jnp.mod(mids_ref[:, 0:1] + par[:, 0:_K], hwf)
    mx = jnp.mod(mids_ref[:, 1:2] + par[:, _K:2 * _K], hwf)
    sig = ((jax.nn.softplus(par[:, 2 * _K:3 * _K] + _SIGMA_BOOST)
            + _MIN_SIGMA) * (hwf * _SIGMA_SCALE))
    fyi = jnp.floor(my).astype(jnp.int32)
    fxi = jnp.floor(mx).astype(jnp.int32)

    iys = []
    ixs = []
    for dy, dx in ((0, 0), (0, 1), (1, 0), (1, 1)):
        iys.append(jnp.mod(fyi + dy, _H))
        ixs.append(jnp.mod(fxi + dx, _W))
    iys.append(giy0_ref[0])
    ixs.append(gix0_ref[0])
    iys.append(giy1_ref[0])
    ixs.append(gix1_ref[0])
    iys.append(jnp.mod(fyi + roy0_ref[0], _H))
    ixs.append(jnp.mod(fxi + rox0_ref[0], _W))
    iys.append(jnp.mod(fyi + roy1_ref[0], _H))
    ixs.append(jnp.mod(fxi + rox1_ref[0], _W))

    ids = [iy * _W + ix for iy, ix in zip(iys, ixs)]
    props = []
    for v in range(_VS):
        dyv = (iys[v].astype(jnp.float32) - my) / sig
        dxv = (ixs[v].astype(jnp.float32) - mx) / sig
        pr = jnp.exp(-0.5 * (dyv * dyv + dxv * dxv))
        if v > 0:
            dup = ids[0] == ids[v]
            for u in range(1, v):
                dup = jnp.logical_or(dup, ids[u] == ids[v])
            pr = jnp.where(dup, 0.0, pr)
        props.append(pr)
    tot = props[0]
    for v in range(1, _VS):
        tot = tot + props[v]
    base = b * _P
    ids_ref[0] = jnp.concatenate([i + base for i in ids], axis=1)
    wts_ref[0] = jnp.concatenate([p / tot for p in props], axis=1)


def _stage_c_body(g_ref, w_ref, wu_ref, bu_ref, out_ref):
    g = g_ref[...]
    wv = w_ref[...]
    feats = []
    for k in range(_K):
        acc = None
        for v in range(_VS):
            j = v * _K + k
            term = g[:, j * _C:(j + 1) * _C] * wv[:, j:j + 1]
            acc = term if acc is None else acc + term
        feats.append(acc)
    f = jnp.concatenate(feats, axis=1)
    out_ref[...] = (jnp.dot(f, wu_ref[...], preferred_element_type=jnp.float32)
                    + bu_ref[...])


def _sc_gather_body(table_hbm, idx_hbm, out_hbm, idx_v, rows_v, sem):
    wid = lax.axis_index("s") * 2 + lax.axis_index("c")
    wbase = wid * _PER_W
    pltpu.sync_copy(idx_hbm.at[pl.ds(wbase, _PER_W)], idx_v)

    def step(i, carry):
        off = i * _CH
        pltpu.async_copy(
            table_hbm.at[idx_v.at[pl.ds(off, _CH)]], rows_v, sem).wait()
        pltpu.sync_copy(rows_v, out_hbm.at[pl.ds(wbase + off, _CH)])
        return carry

    lax.fori_loop(0, _PER_W // _CH, step, 0)


@jax.jit
def kernel(x, W1, b1, W2, b2, Wu, bu):
    b, c, h, w = x.shape
    hwf = jnp.float32(h)

    xt = jnp.transpose(x, (0, 2, 3, 1))
    table = xt.reshape(b * _P, c)
    ys = jnp.linspace(0.0, 1.0, h)
    xs = jnp.linspace(0.0, 1.0, w)
    gy, gx = jnp.meshgrid(ys, xs, indexing="ij")
    coords = jnp.stack([gy, gx], axis=-1).reshape(_P, 2)
    mids = coords * (hwf - 1.0)
    xin = jnp.concatenate(
        [xt.reshape(b, _P, c),
         jnp.broadcast_to(coords[None], (b, _P, 2))], axis=-1)

    perm = jnp.array(
        tuple(range(0, 2 * _K, 2)) + tuple(range(1, 2 * _K, 2))
        + tuple(range(2 * _K, 3 * _K)), dtype=jnp.int32)
    W2p = W2[:, perm]
    b2p = b2[perm].reshape(1, 3 * _K)
    b1r = b1.reshape(1, _HID)
    bur = bu.reshape(1, _COUT)

    kg, kr = jax.random.split(jax.random.key(42))
    gidx = jax.random.randint(kg, (b, h, w, _K, 2, 2), 0, jnp.array([h, w]))
    roff = jax.random.randint(
        kr, (b, h, w, _K, 2, 2), 0,
        jnp.array([_REGION, _REGION])) - jnp.array([_REGION, _REGION]) // 2
    g4 = gidx.reshape(b, _P, _K, 2, 2).astype(jnp.int32)
    r4 = roff.reshape(b, _P, _K, 2, 2).astype(jnp.int32)
    giy0, gix0 = g4[..., 0, 0], g4[..., 0, 1]
    giy1, gix1 = g4[..., 1, 0], g4[..., 1, 1]
    roy0, rox0 = r4[..., 0, 0], r4[..., 0, 1]
    roy1, rox1 = r4[..., 1, 0], r4[..., 1, 1]

    pb = 512
    bspec = pl.BlockSpec((1, pb, _K), lambda i, j: (i, j, 0))
    ids, wts = pl.pallas_call(
        _stage_a_body,
        grid=(b, _P // pb),
        in_specs=[
            pl.BlockSpec((1, pb, _ADIN), lambda i, j: (i, j, 0)),
            pl.BlockSpec((_ADIN, _HID), lambda i, j: (0, 0)),
            pl.BlockSpec((1, _HID), lambda i, j: (0, 0)),
            pl.BlockSpec((_HID, 3 * _K), lambda i, j: (0, 0)),
            pl.BlockSpec((1, 3 * _K), lambda i, j: (0, 0)),
            pl.BlockSpec((pb, 2), lambda i, j: (j, 0)),
            bspec, bspec, bspec, bspec, bspec, bspec, bspec, bspec,
        ],
        out_specs=[
            pl.BlockSpec((1, pb, _VS * _K), lambda i, j: (i, j, 0)),
            pl.BlockSpec((1, pb, _VS * _K), lambda i, j: (i, j, 0)),
        ],
        out_shape=[
            jax.ShapeDtypeStruct((b, _P, _VS * _K), jnp.int32),
            jax.ShapeDtypeStruct((b, _P, _VS * _K), jnp.float32),
        ],
    )(xin, W1, b1r, W2p, b2p, mids,
      giy0, gix0, giy1, gix1, roy0, rox0, roy1, rox1)

    idx_flat = ids.reshape(_NIDX)
    mesh = plsc.VectorSubcoreMesh(core_axis_name="c", subcore_axis_name="s")
    gather = functools.partial(
        pl.kernel, _sc_gather_body, mesh=mesh,
        compiler_params=pltpu.CompilerParams(use_tc_tiling_on_sc=False),
        out_type=jax.ShapeDtypeStruct((_NIDX, _C), jnp.float32),
        scratch_types=[
            pltpu.VMEM((_PER_W,), jnp.int32),
            pltpu.VMEM((_CH, _C), jnp.float32),
            pltpu.SemaphoreType.DMA,
        ],
    )()
    gathered = gather(table, idx_flat)

    pc = 256
    g2 = gathered.reshape(b * _P, _VS * _K * _C)
    w2d = wts.reshape(b * _P, _VS * _K)
    out = pl.pallas_call(
        _stage_c_body,
        grid=(b * _P // pc,),
        in_specs=[
            pl.BlockSpec((pc, _VS * _K * _C), lambda i: (i, 0)),
            pl.BlockSpec((pc, _VS * _K), lambda i: (i, 0)),
            pl.BlockSpec((_K * _C, _COUT), lambda i: (0, 0)),
            pl.BlockSpec((1, _COUT), lambda i: (0, 0)),
        ],
        out_specs=pl.BlockSpec((pc, _COUT), lambda i: (i, 0)),
        out_shape=jax.ShapeDtypeStruct((b * _P, _COUT), jnp.float32),
    )(g2, w2d, Wu, bur)

    return jnp.transpose(out.reshape(b, h, w, _COUT), (0, 3, 1, 2))


# speedup vs baseline: 18.4762x; 1.0616x over previous
import functools

import jax
import jax.numpy as jnp
from jax import lax
from jax.experimental import pallas as pl
from jax.experimental.pallas import tpu as pltpu
from jax.experimental.pallas import tpu_sc as plsc

_B, _C, _H, _W = 4, 32, 64, 64
_K = 8
_VS = 8
_P = _H * _W
_COUT = 64
_HID = 4 * _C
_ADIN = 2 + _C
_NIDX = _B * _P * _K * _VS
_NW = 32
_CH = 2048
_PER_W = _NIDX // _NW
_MIN_SIGMA = 0.05
_SIGMA_SCALE = 0.05
_SIGMA_BOOST = 2.0
_REGION = 12


def _stage_a_body(xin_ref, w1_ref, b1_ref, w2_ref, b2_ref, mids_ref,
                  giy0_ref, gix0_ref, giy1_ref, gix1_ref,
                  roy0_ref, rox0_ref, roy1_ref, rox1_ref,
                  ids_ref, wts_ref):
    b = pl.program_id(0)
    xin = xin_ref[0]
    hid = jnp.maximum(
        jnp.dot(xin, w1_ref[...], preferred_element_type=jnp.float32)
        + b1_ref[...], 0.0)
    par = (jnp.dot(hid, w2_ref[...], preferred_element_type=jnp.float32)
           + b2_ref[...])
    hwf = jnp.float32(_H)
    my = jnp.mod(mids_ref[:, 0:1] + par[:, 0:_K], hwf)
    mx = jnp.mod(mids_ref[:, 1:2] + par[:, _K:2 * _K], hwf)
    sig = ((jax.nn.softplus(par[:, 2 * _K:3 * _K] + _SIGMA_BOOST)
            + _MIN_SIGMA) * (hwf * _SIGMA_SCALE))
    fyi = jnp.floor(my).astype(jnp.int32)
    fxi = jnp.floor(mx).astype(jnp.int32)

    iys = []
    ixs = []
    for dy, dx in ((0, 0), (0, 1), (1, 0), (1, 1)):
        iys.append(jnp.mod(fyi + dy, _H))
        ixs.append(jnp.mod(fxi + dx, _W))
    iys.append(giy0_ref[0])
    ixs.append(gix0_ref[0])
    iys.append(giy1_ref[0])
    ixs.append(gix1_ref[0])
    iys.append(jnp.mod(fyi + roy0_ref[0], _H))
    ixs.append(jnp.mod(fxi + rox0_ref[0], _W))
    iys.append(jnp.mod(fyi + roy1_ref[0], _H))
    ixs.append(jnp.mod(fxi + rox1_ref[0], _W))

    ids = [iy * _W + ix for iy, ix in zip(iys, ixs)]
    props = []
    for v in range(_VS):
        dyv = (iys[v].astype(jnp.float32) - my) / sig
        dxv = (ixs[v].astype(jnp.float32) - mx) / sig
        pr = jnp.exp(-0.5 * (dyv * dyv + dxv * dxv))
        if v > 0:
            dup = ids[0] == ids[v]
            for u in range(1, v):
                dup = jnp.logical_or(dup, ids[u] == ids[v])
            pr = jnp.where(dup, 0.0, pr)
        props.append(pr)
    tot = props[0]
    for v in range(1, _VS):
        tot = tot + props[v]
    base = b * _P
    ids_ref[0] = jnp.concatenate([i + base for i in ids], axis=1)
    wts_ref[0] = jnp.concatenate([p / tot for p in props], axis=1)


def _stage_c_body(g_ref, w_ref, wu_ref, bu_ref, out_ref):
    g = g_ref[...]
    wv = w_ref[...]
    feats = []
    for k in range(_K):
        acc = None
        for v in range(_VS):
            j = v * _K + k
            term = g[:, j * _C:(j + 1) * _C] * wv[:, j:j + 1]
            acc = term if acc is None else acc + term
        feats.append(acc)
    f = jnp.concatenate(feats, axis=1)
    out_ref[...] = (jnp.dot(f, wu_ref[...], preferred_element_type=jnp.float32)
                    + bu_ref[...])


def _sc_gather_body(table_hbm, idx_hbm, out_hbm, idx_v, rows_v, sem):
    wid = lax.axis_index("s") * 2 + lax.axis_index("c")
    wbase = wid * _PER_W
    pltpu.sync_copy(idx_hbm.at[pl.ds(wbase, _PER_W)], idx_v)

    def step(i, carry):
        off = i * _CH
        pltpu.async_copy(
            table_hbm.at[idx_v.at[pl.ds(off, _CH)]], rows_v, sem).wait()
        pltpu.sync_copy(rows_v, out_hbm.at[pl.ds(wbase + off, _CH)])
        return carry

    lax.fori_loop(0, _PER_W // _CH, step, 0)


@jax.jit
def kernel(x, W1, b1, W2, b2, Wu, bu):
    b, c, h, w = x.shape
    hwf = jnp.float32(h)

    xt = jnp.transpose(x, (0, 2, 3, 1))
    table = xt.reshape(b * _P, c)
    ys = jnp.linspace(0.0, 1.0, h)
    xs = jnp.linspace(0.0, 1.0, w)
    gy, gx = jnp.meshgrid(ys, xs, indexing="ij")
    coords = jnp.stack([gy, gx], axis=-1).reshape(_P, 2)
    mids = coords * (hwf - 1.0)
    xin = jnp.concatenate(
        [xt.reshape(b, _P, c),
         jnp.broadcast_to(coords[None], (b, _P, 2))], axis=-1)

    perm = jnp.array(
        tuple(range(0, 2 * _K, 2)) + tuple(range(1, 2 * _K, 2))
        + tuple(range(2 * _K, 3 * _K)), dtype=jnp.int32)
    W2p = W2[:, perm]
    b2p = b2[perm].reshape(1, 3 * _K)
    b1r = b1.reshape(1, _HID)
    bur = bu.reshape(1, _COUT)

    kg, kr = jax.random.split(jax.random.key(42))
    gidx = jax.random.randint(kg, (b, h, w, _K, 2, 2), 0, jnp.array([h, w]))
    roff = jax.random.randint(
        kr, (b, h, w, _K, 2, 2), 0,
        jnp.array([_REGION, _REGION])) - jnp.array([_REGION, _REGION]) // 2
    g4 = gidx.reshape(b, _P, _K, 2, 2).astype(jnp.int32)
    r4 = roff.reshape(b, _P, _K, 2, 2).astype(jnp.int32)
    giy0, gix0 = g4[..., 0, 0], g4[..., 0, 1]
    giy1, gix1 = g4[..., 1, 0], g4[..., 1, 1]
    roy0, rox0 = r4[..., 0, 0], r4[..., 0, 1]
    roy1, rox1 = r4[..., 1, 0], r4[..., 1, 1]

    pb = 512
    bspec = pl.BlockSpec((1, pb, _K), lambda i, j: (i, j, 0))
    ids, wts = pl.pallas_call(
        _stage_a_body,
        grid=(b, _P // pb),
        in_specs=[
            pl.BlockSpec((1, pb, _ADIN), lambda i, j: (i, j, 0)),
            pl.BlockSpec((_ADIN, _HID), lambda i, j: (0, 0)),
            pl.BlockSpec((1, _HID), lambda i, j: (0, 0)),
            pl.BlockSpec((_HID, 3 * _K), lambda i, j: (0, 0)),
            pl.BlockSpec((1, 3 * _K), lambda i, j: (0, 0)),
            pl.BlockSpec((pb, 2), lambda i, j: (j, 0)),
            bspec, bspec, bspec, bspec, bspec, bspec, bspec, bspec,
        ],
        out_specs=[
            pl.BlockSpec((1, pb, _VS * _K), lambda i, j: (i, j, 0)),
            pl.BlockSpec((1, pb, _VS * _K), lambda i, j: (i, j, 0)),
        ],
        out_shape=[
            jax.ShapeDtypeStruct((b, _P, _VS * _K), jnp.int32),
            jax.ShapeDtypeStruct((b, _P, _VS * _K), jnp.float32),
        ],
    )(xin, W1, b1r, W2p, b2p, mids,
      giy0, gix0, giy1, gix1, roy0, rox0, roy1, rox1)

    idx_flat = ids.reshape(_NIDX)
    mesh = plsc.VectorSubcoreMesh(core_axis_name="c", subcore_axis_name="s")
    gather = functools.partial(
        pl.kernel, _sc_gather_body, mesh=mesh,
        compiler_params=pltpu.CompilerParams(use_tc_tiling_on_sc=False),
        out_type=jax.ShapeDtypeStruct((_NIDX, _C), jnp.float32),
        scratch_types=[
            pltpu.VMEM((_PER_W,), jnp.int32),
            pltpu.VMEM((_CH, _C), jnp.float32),
            pltpu.SemaphoreType.DMA,
        ],
    )()
    gathered = gather(table, idx_flat)

    pc = 256
    g2 = gathered.reshape(b * _P, _VS * _K * _C)
    w2d = wts.reshape(b * _P, _VS * _K)
    out = pl.pallas_call(
        _stage_c_body,
        grid=(b * _P // pc,),
        in_specs=[
            pl.BlockSpec((pc, _VS * _K * _C), lambda i: (i, 0)),
            pl.BlockSpec((pc, _VS * _K), lambda i: (i, 0)),
            pl.BlockSpec((_K * _C, _COUT), lambda i: (0, 0)),
            pl.BlockSpec((1, _COUT), lambda i: (0, 0)),
        ],
        out_specs=pl.BlockSpec((pc, _COUT), lambda i: (i, 0)),
        out_shape=jax.ShapeDtypeStruct((b * _P, _COUT), jnp.float32),
    )(g2, w2d, Wu, bur)

    return jnp.transpose(out.reshape(b, h, w, _COUT), (0, 3, 1, 2))


# speedup vs baseline: 18.7744x; 1.0161x over previous
import functools

import jax
import jax.numpy as jnp
from jax import lax
from jax.experimental import pallas as pl
from jax.experimental.pallas import tpu as pltpu
from jax.experimental.pallas import tpu_sc as plsc

_B, _C, _H, _W = 4, 32, 64, 64
_K = 8
_VS = 8
_P = _H * _W
_COUT = 64
_HID = 4 * _C
_ADIN = 2 + _C
_NIDX = _B * _P * _K * _VS
_NW = 32
_CH = 1024
_PER_W = _NIDX // _NW
_MIN_SIGMA = 0.05
_SIGMA_SCALE = 0.05
_SIGMA_BOOST = 2.0
_REGION = 12


def _stage_a_body(xin_ref, w1_ref, b1_ref, w2_ref, b2_ref, mids_ref,
                  giy0_ref, gix0_ref, giy1_ref, gix1_ref,
                  roy0_ref, rox0_ref, roy1_ref, rox1_ref,
                  ids_ref, wts_ref):
    b = pl.program_id(0)
    xin = xin_ref[0]
    hid = jnp.maximum(
        jnp.dot(xin, w1_ref[...], preferred_element_type=jnp.float32)
        + b1_ref[...], 0.0)
    par = (jnp.dot(hid, w2_ref[...], preferred_element_type=jnp.float32)
           + b2_ref[...])
    hwf = jnp.float32(_H)
    my = jnp.mod(mids_ref[:, 0:1] + par[:, 0:_K], hwf)
    mx = jnp.mod(mids_ref[:, 1:2] + par[:, _K:2 * _K], hwf)
    sig = ((jax.nn.softplus(par[:, 2 * _K:3 * _K] + _SIGMA_BOOST)
            + _MIN_SIGMA) * (hwf * _SIGMA_SCALE))
    fyi = jnp.floor(my).astype(jnp.int32)
    fxi = jnp.floor(mx).astype(jnp.int32)

    iys = []
    ixs = []
    for dy, dx in ((0, 0), (0, 1), (1, 0), (1, 1)):
        iys.append(jnp.mod(fyi + dy, _H))
        ixs.append(jnp.mod(fxi + dx, _W))
    iys.append(giy0_ref[0])
    ixs.append(gix0_ref[0])
    iys.append(giy1_ref[0])
    ixs.append(gix1_ref[0])
    iys.append(jnp.mod(fyi + roy0_ref[0], _H))
    ixs.append(jnp.mod(fxi + rox0_ref[0], _W))
    iys.append(jnp.mod(fyi + roy1_ref[0], _H))
    ixs.append(jnp.mod(fxi + rox1_ref[0], _W))

    ids = [iy * _W + ix for iy, ix in zip(iys, ixs)]
    props = []
    for v in range(_VS):
        dyv = (iys[v].astype(jnp.float32) - my) / sig
        dxv = (ixs[v].astype(jnp.float32) - mx) / sig
        pr = jnp.exp(-0.5 * (dyv * dyv + dxv * dxv))
        if v > 0:
            dup = ids[0] == ids[v]
            for u in range(1, v):
                dup = jnp.logical_or(dup, ids[u] == ids[v])
            pr = jnp.where(dup, 0.0, pr)
        props.append(pr)
    tot = props[0]
    for v in range(1, _VS):
        tot = tot + props[v]
    base = b * _P
    ids_ref[0] = jnp.concatenate([i + base for i in ids], axis=1)
    wts_ref[0] = jnp.concatenate([p / tot for p in props], axis=1)


def _stage_c_body(g_ref, w_ref, wu_ref, bu_ref, out_ref):
    g = g_ref[...]
    wv = w_ref[...]
    feats = []
    for k in range(_K):
        acc = None
        for v in range(_VS):
            j = v * _K + k
            term = g[:, j * _C:(j + 1) * _C] * wv[:, j:j + 1]
            acc = term if acc is None else acc + term
        feats.append(acc)
    f = jnp.concatenate(feats, axis=1)
    out_ref[...] = (jnp.dot(f, wu_ref[...], preferred_element_type=jnp.float32)
                    + bu_ref[...])


def _sc_gather_body(table_hbm, idx_hbm, out_hbm, idx_v, rows0, rows1,
                    sem0, sem1):
    wid = lax.axis_index("s") * 2 + lax.axis_index("c")
    wbase = wid * _PER_W
    n = _PER_W // _CH
    bufs = (rows0, rows1)
    sems = (sem0, sem1)
    pltpu.sync_copy(idx_hbm.at[pl.ds(wbase, _PER_W)], idx_v)

    def start(i, t):
        pltpu.async_copy(
            table_hbm.at[idx_v.at[pl.ds(i * _CH, _CH)]], bufs[t], sems[t])

    def finish(i, t):
        pltpu.make_async_copy(
            table_hbm.at[idx_v.at[pl.ds(i * _CH, _CH)]], bufs[t],
            sems[t]).wait()
        pltpu.sync_copy(bufs[t], out_hbm.at[pl.ds(wbase + i * _CH, _CH)])

    start(0, 0)
    start(1, 1)

    def step(j, carry):
        for t in range(2):
            i = 2 * j + t
            finish(i, t)
            start(i + 2, t)
        return carry

    lax.fori_loop(0, n // 2 - 1, step, 0)
    finish(n - 2, 0)
    finish(n - 1, 1)


@jax.jit
def kernel(x, W1, b1, W2, b2, Wu, bu):
    b, c, h, w = x.shape
    hwf = jnp.float32(h)

    xt = jnp.transpose(x, (0, 2, 3, 1))
    table = xt.reshape(b * _P, c)
    ys = jnp.linspace(0.0, 1.0, h)
    xs = jnp.linspace(0.0, 1.0, w)
    gy, gx = jnp.meshgrid(ys, xs, indexing="ij")
    coords = jnp.stack([gy, gx], axis=-1).reshape(_P, 2)
    mids = coords * (hwf - 1.0)
    xin = jnp.concatenate(
        [xt.reshape(b, _P, c),
         jnp.broadcast_to(coords[None], (b, _P, 2))], axis=-1)

    perm = jnp.array(
        tuple(range(0, 2 * _K, 2)) + tuple(range(1, 2 * _K, 2))
        + tuple(range(2 * _K, 3 * _K)), dtype=jnp.int32)
    W2p = W2[:, perm]
    b2p = b2[perm].reshape(1, 3 * _K)
    b1r = b1.reshape(1, _HID)
    bur = bu.reshape(1, _COUT)

    kg, kr = jax.random.split(jax.random.key(42))
    gidx = jax.random.randint(kg, (b, h, w, _K, 2, 2), 0, jnp.array([h, w]))
    roff = jax.random.randint(
        kr, (b, h, w, _K, 2, 2), 0,
        jnp.array([_REGION, _REGION])) - jnp.array([_REGION, _REGION]) // 2
    g4 = gidx.reshape(b, _P, _K, 2, 2).astype(jnp.int32)
    r4 = roff.reshape(b, _P, _K, 2, 2).astype(jnp.int32)
    giy0, gix0 = g4[..., 0, 0], g4[..., 0, 1]
    giy1, gix1 = g4[..., 1, 0], g4[..., 1, 1]
    roy0, rox0 = r4[..., 0, 0], r4[..., 0, 1]
    roy1, rox1 = r4[..., 1, 0], r4[..., 1, 1]

    pb = 512
    bspec = pl.BlockSpec((1, pb, _K), lambda i, j: (i, j, 0))
    ids, wts = pl.pallas_call(
        _stage_a_body,
        grid=(b, _P // pb),
        in_specs=[
            pl.BlockSpec((1, pb, _ADIN), lambda i, j: (i, j, 0)),
            pl.BlockSpec((_ADIN, _HID), lambda i, j: (0, 0)),
            pl.BlockSpec((1, _HID), lambda i, j: (0, 0)),
            pl.BlockSpec((_HID, 3 * _K), lambda i, j: (0, 0)),
            pl.BlockSpec((1, 3 * _K), lambda i, j: (0, 0)),
            pl.BlockSpec((pb, 2), lambda i, j: (j, 0)),
            bspec, bspec, bspec, bspec, bspec, bspec, bspec, bspec,
        ],
        out_specs=[
            pl.BlockSpec((1, pb, _VS * _K), lambda i, j: (i, j, 0)),
            pl.BlockSpec((1, pb, _VS * _K), lambda i, j: (i, j, 0)),
        ],
        out_shape=[
            jax.ShapeDtypeStruct((b, _P, _VS * _K), jnp.int32),
            jax.ShapeDtypeStruct((b, _P, _VS * _K), jnp.float32),
        ],
    )(xin, W1, b1r, W2p, b2p, mids,
      giy0, gix0, giy1, gix1, roy0, rox0, roy1, rox1)

    idx_flat = ids.reshape(_NIDX)
    mesh = plsc.VectorSubcoreMesh(core_axis_name="c", subcore_axis_name="s")
    gather = functools.partial(
        pl.kernel, _sc_gather_body, mesh=mesh,
        compiler_params=pltpu.CompilerParams(use_tc_tiling_on_sc=False),
        out_type=jax.ShapeDtypeStruct((_NIDX, _C), jnp.float32),
        scratch_types=[
            pltpu.VMEM((_PER_W,), jnp.int32),
            pltpu.VMEM((_CH, _C), jnp.float32),
            pltpu.VMEM((_CH, _C), jnp.float32),
            pltpu.SemaphoreType.DMA,
            pltpu.SemaphoreType.DMA,
        ],
    )()
    gathered = gather(table, idx_flat)

    pc = 256
    g2 = gathered.reshape(b * _P, _VS * _K * _C)
    w2d = wts.reshape(b * _P, _VS * _K)
    out = pl.pallas_call(
        _stage_c_body,
        grid=(b * _P // pc,),
        in_specs=[
            pl.BlockSpec((pc, _VS * _K * _C), lambda i: (i, 0)),
            pl.BlockSpec((pc, _VS * _K), lambda i: (i, 0)),
            pl.BlockSpec((_K * _C, _COUT), lambda i: (0, 0)),
            pl.BlockSpec((1, _COUT), lambda i: (0, 0)),
        ],
        out_specs=pl.BlockSpec((pc, _COUT), lambda i: (i, 0)),
        out_shape=jax.ShapeDtypeStruct((b * _P, _COUT), jnp.float32),
    )(g2, w2d, Wu, bur)

    return jnp.transpose(out.reshape(b, h, w, _COUT), (0, 3, 1, 2))
